# Initial kernel scaffold; baseline (speedup 1.0000x reference)
import jax
import jax.numpy as jnp
from jax.experimental import pallas as pl


def kernel(h, edge_index):
    raise NotImplementedError("write your pallas kernel here")


# speedup vs baseline: 1.4062x; 1.4062x over previous
import functools

import jax
import jax.numpy as jnp
from jax import lax
from jax.experimental import pallas as pl
from jax.experimental.pallas import tpu as pltpu
from jax.experimental.pallas import tpu_sc as plsc

N_NODES = 10000
N_EDGES = 160000
D_FEAT = 256

NC = 2
NS = 16
NW = NC * NS
LANES = 16

CHUNK = 128
NCHUNK = 40
E_PER_W = CHUNK * NCHUNK
E_PAD = E_PER_W * NW


TPAD = LANES + 1


def _sc_body(h_hbm, src_hbm, dst_hbm, out_hbm,
             src_v, dst_v, u_v, v_v, out_v, tr_v, sem):
    wid = lax.axis_index("s") * NC + lax.axis_index("c")
    pltpu.sync_copy(src_hbm.at[wid], src_v)
    pltpu.sync_copy(dst_hbm.at[wid], dst_v)

    lane = jax.lax.iota(jnp.int32, LANES)

    def chunk_body(c, carry):
        cp_u = pltpu.async_copy(h_hbm.at[src_v.at[c]], u_v, sem)
        cp_v = pltpu.async_copy(h_hbm.at[dst_v.at[c]], v_v, sem)
        cp_u.wait()
        cp_v.wait()

        def group_body(g, carry2):
            def edge_body(t, carry3):
                e = g * LANES + t
                acc = u_v[e, pl.ds(0, LANES)] * v_v[e, pl.ds(0, LANES)]
                for j in range(1, D_FEAT // LANES):
                    acc = acc + (u_v[e, pl.ds(j * LANES, LANES)] *
                                 v_v[e, pl.ds(j * LANES, LANES)])
                tr_v[pl.ds(t * TPAD, LANES)] = acc
                return carry3

            lax.fori_loop(0, LANES, edge_body, 0, unroll=False)

            res = plsc.load_gather(tr_v, [lane * TPAD])
            for j in range(1, LANES):
                res = res + plsc.load_gather(tr_v, [lane * TPAD + j])
            out_v[pl.ds(c * CHUNK + g * LANES, LANES)] = res
            return carry2

        lax.fori_loop(0, CHUNK // LANES, group_body, 0, unroll=False)
        return carry

    lax.fori_loop(0, NCHUNK, chunk_body, 0, unroll=False)
    pltpu.sync_copy(out_v, out_hbm.at[pl.ds(wid * E_PER_W, E_PER_W)])


@jax.jit
def _dot_scores(h, src, dst):
    mesh = plsc.VectorSubcoreMesh(core_axis_name="c", subcore_axis_name="s")
    kern = functools.partial(
        pl.kernel,
        mesh=mesh,
        out_type=jax.ShapeDtypeStruct((E_PAD,), jnp.float32),
        scratch_types=[
            pltpu.VMEM((NCHUNK, CHUNK), jnp.int32),
            pltpu.VMEM((NCHUNK, CHUNK), jnp.int32),
            pltpu.VMEM((CHUNK, D_FEAT), jnp.float32),
            pltpu.VMEM((CHUNK, D_FEAT), jnp.float32),
            pltpu.VMEM((E_PER_W,), jnp.float32),
            pltpu.VMEM((LANES * TPAD,), jnp.float32),
            pltpu.SemaphoreType.DMA,
        ],
        compiler_params=pltpu.CompilerParams(needs_layout_passes=False),
    )(_sc_body)
    return kern(h, src, dst)


def kernel(h, edge_index):
    src = edge_index[0].astype(jnp.int32)
    dst = edge_index[1].astype(jnp.int32)
    pad = E_PAD - N_EDGES
    src = jnp.concatenate([src, jnp.zeros((pad,), jnp.int32)])
    dst = jnp.concatenate([dst, jnp.zeros((pad,), jnp.int32)])
    src = src.reshape(NW, NCHUNK, CHUNK)
    dst = dst.reshape(NW, NCHUNK, CHUNK)
    scores = _dot_scores(h, src, dst)
    return scores[:N_EDGES]


# speedup vs baseline: 1.6722x; 1.1891x over previous
import functools

import jax
import jax.numpy as jnp
from jax import lax
from jax.experimental import pallas as pl
from jax.experimental.pallas import tpu as pltpu
from jax.experimental.pallas import tpu_sc as plsc

N_NODES = 10000
N_EDGES = 160000
D_FEAT = 256

NC = 2
NS = 16
NW = NC * NS
LANES = 16

CHUNK = 80
NCHUNK = 64
E_PER_W = CHUNK * NCHUNK
E_PAD = E_PER_W * NW

TPAD = LANES + 1


def _sc_body(h_hbm, src_hbm, dst_hbm, out_hbm,
             src_v, dst_v, u0_v, v0_v, u1_v, v1_v, out_v, tr_v,
             sem0, sem1):
    wid = lax.axis_index("s") * NC + lax.axis_index("c")
    pltpu.sync_copy(src_hbm.at[wid], src_v)
    pltpu.sync_copy(dst_hbm.at[wid], dst_v)

    lane = jax.lax.iota(jnp.int32, LANES)

    def issue(c, u_v, v_v, sem):
        pltpu.async_copy(h_hbm.at[src_v.at[c]], u_v, sem)
        pltpu.async_copy(h_hbm.at[dst_v.at[c]], v_v, sem)

    def drain(u_v, v_v, sem):
        pltpu.make_async_copy(h_hbm.at[pl.ds(0, CHUNK)], u_v, sem).wait()
        pltpu.make_async_copy(h_hbm.at[pl.ds(0, CHUNK)], v_v, sem).wait()

    def compute(c, u_v, v_v):
        def group_body(g, carry2):
            def edge_body(t, carry3):
                e = g * LANES + t
                acc = u_v[e, pl.ds(0, LANES)] * v_v[e, pl.ds(0, LANES)]
                for j in range(1, D_FEAT // LANES):
                    acc = acc + (u_v[e, pl.ds(j * LANES, LANES)] *
                                 v_v[e, pl.ds(j * LANES, LANES)])
                tr_v[pl.ds(t * TPAD, LANES)] = acc
                return carry3

            lax.fori_loop(0, LANES, edge_body, 0, unroll=False)

            res = plsc.load_gather(tr_v, [lane * TPAD])
            for j in range(1, LANES):
                res = res + plsc.load_gather(tr_v, [lane * TPAD + j])
            out_v[pl.ds(c * CHUNK + g * LANES, LANES)] = res
            return carry2

        lax.fori_loop(0, CHUNK // LANES, group_body, 0, unroll=False)

    issue(0, u0_v, v0_v, sem0)

    def pipe_body(i, carry):
        c0 = 2 * i
        issue(c0 + 1, u1_v, v1_v, sem1)
        drain(u0_v, v0_v, sem0)
        compute(c0, u0_v, v0_v)

        @pl.when(i < (NCHUNK // 2) - 1)
        def _():
            issue(c0 + 2, u0_v, v0_v, sem0)

        drain(u1_v, v1_v, sem1)
        compute(c0 + 1, u1_v, v1_v)
        return carry

    lax.fori_loop(0, NCHUNK // 2, pipe_body, 0, unroll=False)
    pltpu.sync_copy(out_v, out_hbm.at[pl.ds(wid * E_PER_W, E_PER_W)])


@jax.jit
def _dot_scores(h, src, dst):
    mesh = plsc.VectorSubcoreMesh(core_axis_name="c", subcore_axis_name="s")
    kern = functools.partial(
        pl.kernel,
        mesh=mesh,
        out_type=jax.ShapeDtypeStruct((E_PAD,), jnp.float32),
        scratch_types=[
            pltpu.VMEM((NCHUNK, CHUNK), jnp.int32),
            pltpu.VMEM((NCHUNK, CHUNK), jnp.int32),
            pltpu.VMEM((CHUNK, D_FEAT), jnp.float32),
            pltpu.VMEM((CHUNK, D_FEAT), jnp.float32),
            pltpu.VMEM((CHUNK, D_FEAT), jnp.float32),
            pltpu.VMEM((CHUNK, D_FEAT), jnp.float32),
            pltpu.VMEM((E_PER_W,), jnp.float32),
            pltpu.VMEM((LANES * TPAD,), jnp.float32),
            pltpu.SemaphoreType.DMA,
            pltpu.SemaphoreType.DMA,
        ],
        compiler_params=pltpu.CompilerParams(needs_layout_passes=False),
    )(_sc_body)
    return kern(h, src, dst)


def kernel(h, edge_index):
    src = edge_index[0].astype(jnp.int32)
    dst = edge_index[1].astype(jnp.int32)
    pad = E_PAD - N_EDGES
    src = jnp.concatenate([src, jnp.zeros((pad,), jnp.int32)])
    dst = jnp.concatenate([dst, jnp.zeros((pad,), jnp.int32)])
    src = src.reshape(NW, NCHUNK, CHUNK)
    dst = dst.reshape(NW, NCHUNK, CHUNK)
    scores = _dot_scores(h, src, dst)
    return scores[:N_EDGES]


# speedup vs baseline: 3.7654x; 2.2518x over previous
import functools

import jax
import jax.numpy as jnp
from jax import lax
from jax.experimental import pallas as pl
from jax.experimental.pallas import tpu as pltpu
from jax.experimental.pallas import tpu_sc as plsc

N_NODES = 10000
N_EDGES = 160000
D_FEAT = 256

NC = 2
NS = 16
LANES = 16

CHUNK = 80
NCHUNK_TOT = 125
N_FAST = 94
N_SLOW = NCHUNK_TOT - N_FAST
FAST_CORE = 0

TPAD = LANES + 1


def _sc_body(h_hbm, src_hbm, dst_hbm, out_hbm,
             src_v, dst_v, u0_v, v0_v, u1_v, v1_v, out_v, tr_v,
             sem0, sem1):
    cid = lax.axis_index("c")
    sid = lax.axis_index("s")

    lane = jax.lax.iota(jnp.int32, LANES)

    def issue(lc, u_v, v_v, sem):
        pltpu.async_copy(
            h_hbm.at[src_v.at[pl.ds(lc * CHUNK, CHUNK)]], u_v, sem)
        pltpu.async_copy(
            h_hbm.at[dst_v.at[pl.ds(lc * CHUNK, CHUNK)]], v_v, sem)

    def drain(u_v, v_v, sem):
        pltpu.make_async_copy(h_hbm.at[pl.ds(0, CHUNK)], u_v, sem).wait()
        pltpu.make_async_copy(h_hbm.at[pl.ds(0, CHUNK)], v_v, sem).wait()

    def compute(lc, u_v, v_v):
        def group_body(g, carry2):
            def edge_body(t, carry3):
                e = g * LANES + t
                acc = u_v[e, pl.ds(0, LANES)] * v_v[e, pl.ds(0, LANES)]
                for j in range(1, D_FEAT // LANES):
                    acc = acc + (u_v[e, pl.ds(j * LANES, LANES)] *
                                 v_v[e, pl.ds(j * LANES, LANES)])
                tr_v[pl.ds(t * TPAD, LANES)] = acc
                return carry3

            lax.fori_loop(0, LANES, edge_body, 0, unroll=False)

            res = plsc.load_gather(tr_v, [lane * TPAD])
            for j in range(1, LANES):
                res = res + plsc.load_gather(tr_v, [lane * TPAD + j])
            out_v[pl.ds(lc * CHUNK + g * LANES, LANES)] = res
            return carry2

        lax.fori_loop(0, CHUNK // LANES, group_body, 0, unroll=False)

    def run(cstart, n):
        ne = n * CHUNK
        base = cstart * CHUNK
        pltpu.sync_copy(src_hbm.at[pl.ds(base, ne)], src_v.at[pl.ds(0, ne)])
        pltpu.sync_copy(dst_hbm.at[pl.ds(base, ne)], dst_v.at[pl.ds(0, ne)])

        half = n // 2
        issue(0, u0_v, v0_v, sem0)

        def pipe_body(i, carry):
            c0 = 2 * i
            issue(c0 + 1, u1_v, v1_v, sem1)
            drain(u0_v, v0_v, sem0)
            compute(c0, u0_v, v0_v)

            if n % 2 == 1:
                issue(c0 + 2, u0_v, v0_v, sem0)
            else:
                @pl.when(i < half - 1)
                def _():
                    issue(c0 + 2, u0_v, v0_v, sem0)

            drain(u1_v, v1_v, sem1)
            compute(c0 + 1, u1_v, v1_v)
            return carry

        lax.fori_loop(0, half, pipe_body, 0, unroll=False)

        if n % 2 == 1:
            drain(u0_v, v0_v, sem0)
            compute(n - 1, u0_v, v0_v)

        pltpu.sync_copy(
            out_v.at[pl.ds(0, n * CHUNK)],
            out_hbm.at[pl.ds(cstart * CHUNK, n * CHUNK)],
        )

    @pl.when(cid == FAST_CORE)
    def _():
        run(sid * N_FAST, N_FAST)

    @pl.when(cid == 1 - FAST_CORE)
    def _():
        run(NS * N_FAST + sid * N_SLOW, N_SLOW)


@jax.jit
def _dot_scores(h, src, dst):
    mesh = plsc.VectorSubcoreMesh(core_axis_name="c", subcore_axis_name="s")
    kern = functools.partial(
        pl.kernel,
        mesh=mesh,
        out_type=jax.ShapeDtypeStruct((N_EDGES,), jnp.float32),
        scratch_types=[
            pltpu.VMEM((N_FAST * CHUNK,), jnp.int32),
            pltpu.VMEM((N_FAST * CHUNK,), jnp.int32),
            pltpu.VMEM((CHUNK, D_FEAT), jnp.float32),
            pltpu.VMEM((CHUNK, D_FEAT), jnp.float32),
            pltpu.VMEM((CHUNK, D_FEAT), jnp.float32),
            pltpu.VMEM((CHUNK, D_FEAT), jnp.float32),
            pltpu.VMEM((N_FAST * CHUNK,), jnp.float32),
            pltpu.VMEM((LANES * TPAD,), jnp.float32),
            pltpu.SemaphoreType.DMA,
            pltpu.SemaphoreType.DMA,
        ],
        compiler_params=pltpu.CompilerParams(needs_layout_passes=False),
    )(_sc_body)
    return kern(h, src, dst)


def kernel(h, edge_index):
    src = edge_index[0].astype(jnp.int32)
    dst = edge_index[1].astype(jnp.int32)
    return _dot_scores(h, src, dst)


# speedup vs baseline: 4.7438x; 1.2598x over previous
import functools

import jax
import jax.numpy as jnp
from jax import lax
from jax.experimental import pallas as pl
from jax.experimental.pallas import tpu as pltpu
from jax.experimental.pallas import tpu_sc as plsc

N_NODES = 10000
N_EDGES = 160000
D_FEAT = 256

NC = 2
NS = 16
LANES = 16

CHUNK = 80
NCHUNK_TOT = 125
N_FAST = 65
N_SLOW = NCHUNK_TOT - N_FAST
FAST_CORE = 0

TPAD = LANES + 1


def _sc_body(h_hbm, src_hbm, dst_hbm, out_hbm,
             src_v, dst_v, u0_v, v0_v, u1_v, v1_v, out_v, tr_v,
             sem0, sem1):
    cid = lax.axis_index("c")
    sid = lax.axis_index("s")

    lane = jax.lax.iota(jnp.int32, LANES)

    def issue(lc, u_v, v_v, sem):
        pltpu.async_copy(
            h_hbm.at[src_v.at[pl.ds(lc * CHUNK, CHUNK)]], u_v, sem)
        pltpu.async_copy(
            h_hbm.at[dst_v.at[pl.ds(lc * CHUNK, CHUNK)]], v_v, sem)

    def drain(u_v, v_v, sem):
        pltpu.make_async_copy(h_hbm.at[pl.ds(0, CHUNK)], u_v, sem).wait()
        pltpu.make_async_copy(h_hbm.at[pl.ds(0, CHUNK)], v_v, sem).wait()

    def compute(lc, u_v, v_v):
        def group_body(g, carry2):
            def edge_body(t, carry3):
                e = g * LANES + t
                accs = [u_v[e, pl.ds(j * LANES, LANES)] *
                        v_v[e, pl.ds(j * LANES, LANES)]
                        for j in range(4)]
                for j in range(4, D_FEAT // LANES):
                    accs[j % 4] = accs[j % 4] + (
                        u_v[e, pl.ds(j * LANES, LANES)] *
                        v_v[e, pl.ds(j * LANES, LANES)])
                acc = (accs[0] + accs[1]) + (accs[2] + accs[3])
                tr_v[pl.ds(t * TPAD, LANES)] = acc
                return carry3

            lax.fori_loop(0, LANES, edge_body, 0, unroll=2)

            res = plsc.load_gather(tr_v, [lane * TPAD])
            for j in range(1, LANES):
                res = res + plsc.load_gather(tr_v, [lane * TPAD + j])
            out_v[pl.ds(lc * CHUNK + g * LANES, LANES)] = res
            return carry2

        lax.fori_loop(0, CHUNK // LANES, group_body, 0, unroll=False)

    def run(cstart, n):
        ne = n * CHUNK
        base = cstart * CHUNK
        pltpu.sync_copy(src_hbm.at[pl.ds(base, ne)], src_v.at[pl.ds(0, ne)])
        pltpu.sync_copy(dst_hbm.at[pl.ds(base, ne)], dst_v.at[pl.ds(0, ne)])

        half = n // 2
        issue(0, u0_v, v0_v, sem0)

        def pipe_body(i, carry):
            c0 = 2 * i
            issue(c0 + 1, u1_v, v1_v, sem1)
            drain(u0_v, v0_v, sem0)
            compute(c0, u0_v, v0_v)

            if n % 2 == 1:
                issue(c0 + 2, u0_v, v0_v, sem0)
            else:
                @pl.when(i < half - 1)
                def _():
                    issue(c0 + 2, u0_v, v0_v, sem0)

            drain(u1_v, v1_v, sem1)
            compute(c0 + 1, u1_v, v1_v)
            return carry

        lax.fori_loop(0, half, pipe_body, 0, unroll=False)

        if n % 2 == 1:
            drain(u0_v, v0_v, sem0)
            compute(n - 1, u0_v, v0_v)

        pltpu.sync_copy(
            out_v.at[pl.ds(0, n * CHUNK)],
            out_hbm.at[pl.ds(cstart * CHUNK, n * CHUNK)],
        )

    @pl.when(cid == FAST_CORE)
    def _():
        run(sid * N_FAST, N_FAST)

    @pl.when(cid == 1 - FAST_CORE)
    def _():
        run(NS * N_FAST + sid * N_SLOW, N_SLOW)


@jax.jit
def _dot_scores(h, src, dst):
    mesh = plsc.VectorSubcoreMesh(core_axis_name="c", subcore_axis_name="s")
    kern = functools.partial(
        pl.kernel,
        mesh=mesh,
        out_type=jax.ShapeDtypeStruct((N_EDGES,), jnp.float32),
        scratch_types=[
            pltpu.VMEM((N_FAST * CHUNK,), jnp.int32),
            pltpu.VMEM((N_FAST * CHUNK,), jnp.int32),
            pltpu.VMEM((CHUNK, D_FEAT), jnp.float32),
            pltpu.VMEM((CHUNK, D_FEAT), jnp.float32),
            pltpu.VMEM((CHUNK, D_FEAT), jnp.float32),
            pltpu.VMEM((CHUNK, D_FEAT), jnp.float32),
            pltpu.VMEM((N_FAST * CHUNK,), jnp.float32),
            pltpu.VMEM((LANES * TPAD,), jnp.float32),
            pltpu.SemaphoreType.DMA,
            pltpu.SemaphoreType.DMA,
        ],
        compiler_params=pltpu.CompilerParams(needs_layout_passes=False),
    )(_sc_body)
    return kern(h, src, dst)


def kernel(h, edge_index):
    src = edge_index[0].astype(jnp.int32)
    dst = edge_index[1].astype(jnp.int32)
    return _dot_scores(h, src, dst)
